# Initial kernel scaffold; baseline (speedup 1.0000x reference)
import jax
import jax.numpy as jnp
from jax.experimental import pallas as pl


def kernel(x0, x1, edge_index_101, edge_attr_101, edge_index_110, edge_attr_110, edge_index_021, edge_attr_021, edge_index_030, edge_attr_030, x_emb1, x_emb2, edge_emb1, edge_emb2, gin_W1, gin_b1, gin_W2, gin_b2, W110, b110, W021, b021, W030, b030, bn_gamma, bn_beta):
    raise NotImplementedError("write your pallas kernel here")


# speedup vs baseline: 2.0204x; 2.0204x over previous
import functools

import jax
import jax.numpy as jnp
from jax import lax
from jax.experimental import pallas as pl
from jax.experimental.pallas import tpu as pltpu
from jax.experimental.pallas import tpu_sc as plsc

N = 10000
D = 128
E = 160000
NTILE = 16
CH = 80
GCH = 16
EP = NTILE * CH * 128
CI = 5
NP = NTILE * CI * 128
ACC_ROWS = N + 16
RPT = 624
WB = 104
TAIL = N - NTILE * RPT

_mesh = plsc.VectorSubcoreMesh(core_axis_name="c", subcore_axis_name="s")
_f32 = jnp.float32
_i32 = jnp.int32


def _make_jobs(tile, src_v, dst_v, rows_a, rows_b, zbuf, acc, sem_a, sem_b):
  def init_job(table, fidx, outp):
    pltpu.sync_copy(fidx.at[tile], src_v.at[pl.ds(0, CI)])

    def body(j, carry):
      pltpu.async_copy(table.at[src_v.at[j]], rows_a, sem_a).wait()
      pltpu.sync_copy(rows_a, outp.at[pl.ds(tile * CI * 128 + j * 128, 128)])
      return carry

    lax.fori_loop(0, CI, body, 0)

  def seg_job(table, s2, d2, outp, base):
    if base is None:
      def zb(k, carry):
        pltpu.sync_copy(zbuf, acc.at[pl.ds(tile * RPT + k * 16, 16)])
        return carry
      lax.fori_loop(0, RPT // 16, zb, 0)

      @pl.when(tile == 0)
      def _():
        pltpu.sync_copy(zbuf, acc.at[pl.ds(NTILE * RPT, 16)])
    else:
      def ib(k, carry):
        off = tile * RPT + k * WB
        pltpu.sync_copy(base.at[pl.ds(off, WB)], rows_a.at[pl.ds(0, WB)])
        pltpu.sync_copy(rows_a.at[pl.ds(0, WB)], acc.at[pl.ds(off, WB)])
        return carry
      lax.fori_loop(0, RPT // WB, ib, 0)

      @pl.when(tile == 0)
      def _():
        pltpu.sync_copy(base.at[pl.ds(NTILE * RPT, TAIL)],
                        rows_a.at[pl.ds(0, TAIL)])
        pltpu.sync_copy(rows_a.at[pl.ds(0, TAIL)],
                        acc.at[pl.ds(NTILE * RPT, TAIL)])

    plsc.subcore_barrier()

    def grp(g, carry):
      pltpu.sync_copy(s2.at[tile, pl.ds(g * GCH, GCH)], src_v)
      pltpu.sync_copy(d2.at[tile, pl.ds(g * GCH, GCH)], dst_v)

      def eb(p, c2):
        j = 2 * p
        c0 = pltpu.async_copy(table.at[src_v.at[j]], rows_a, sem_a)
        c1 = pltpu.async_copy(table.at[src_v.at[j + 1]], rows_b, sem_b)
        c0.wait()
        pltpu.sync_copy(rows_a, acc.at[dst_v.at[j]], add=True)
        c1.wait()
        pltpu.sync_copy(rows_b, acc.at[dst_v.at[j + 1]], add=True)
        return c2

      lax.fori_loop(0, GCH // 2, eb, 0)
      return carry

    lax.fori_loop(0, CH // GCH, grp, 0)
    plsc.subcore_barrier()

    def ob(k, carry):
      off = tile * RPT + k * WB
      pltpu.sync_copy(acc.at[pl.ds(off, WB)], rows_a.at[pl.ds(0, WB)])
      pltpu.sync_copy(rows_a.at[pl.ds(0, WB)], outp.at[pl.ds(off, WB)])
      return carry

    lax.fori_loop(0, RPT // WB, ob, 0)

    @pl.when(tile == 0)
    def _():
      pltpu.sync_copy(acc.at[pl.ds(NTILE * RPT, TAIL)],
                      rows_b.at[pl.ds(0, TAIL)])
      pltpu.sync_copy(rows_b.at[pl.ds(0, TAIL)],
                      outp.at[pl.ds(NTILE * RPT, TAIL)])

  return init_job, seg_job


def _zero_zbuf(zbuf):
  z = jnp.zeros((16,), _f32)
  for r in range(16):
    for c in range(8):
      zbuf[r, pl.ds(16 * c, 16)] = z


def _pre_body(t360, t18, f0i, f1i, fa101, dd101, fa021, dd021, fa110, dd110,
              fa030, dd030, h0p, h1p, p101, p021, p110, p030,
              src_v, dst_v, rows_a, rows_b, zbuf, acc, sem_a, sem_b):
  tile = lax.axis_index("s")
  core = lax.axis_index("c")
  _zero_zbuf(zbuf)
  init_job, seg_job = _make_jobs(tile, src_v, dst_v, rows_a, rows_b, zbuf,
                                 acc, sem_a, sem_b)

  @pl.when(core == 0)
  def _():
    init_job(t360, f0i, h0p)
    seg_job(t18, fa101, dd101, p101, None)
    seg_job(t18, fa030, dd030, p030, None)

  @pl.when(core == 1)
  def _():
    init_job(t360, f1i, h1p)
    seg_job(t18, fa021, dd021, p021, None)
    seg_job(t18, fa110, dd110, p110, None)


def _layer_body(h0, h1, q101, q021, q110, q030, s101, dd101, s021, dd021,
                s110, dd110, s030, dd030, a101, a021, a110, a030,
                src_v, dst_v, rows_a, rows_b, zbuf, acc, sem_a, sem_b):
  tile = lax.axis_index("s")
  core = lax.axis_index("c")
  _, seg_job = _make_jobs(tile, src_v, dst_v, rows_a, rows_b, zbuf,
                          acc, sem_a, sem_b)

  @pl.when(core == 0)
  def _():
    seg_job(h1, s101, dd101, a101, q101)
    seg_job(h0, s030, dd030, a030, q030)

  @pl.when(core == 1)
  def _():
    seg_job(h0, s021, dd021, a021, q021)
    seg_job(h1, s110, dd110, a110, q110)


_SC_SCRATCH = [
    pltpu.VMEM((GCH, 128), _i32),
    pltpu.VMEM((GCH, 128), _i32),
    pltpu.VMEM((128, D), _f32),
    pltpu.VMEM((128, D), _f32),
    pltpu.VMEM((16, D), _f32),
    pltpu.VMEM_SHARED((ACC_ROWS, D), _f32),
    pltpu.SemaphoreType.DMA,
    pltpu.SemaphoreType.DMA,
]

_sc_pre = pl.kernel(
    _pre_body,
    out_type=[jax.ShapeDtypeStruct((NP, D), _f32),
              jax.ShapeDtypeStruct((NP, D), _f32)] +
             [jax.ShapeDtypeStruct((N, D), _f32)] * 4,
    mesh=_mesh,
    scratch_types=_SC_SCRATCH,
)

_sc_layer = pl.kernel(
    _layer_body,
    out_type=[jax.ShapeDtypeStruct((N, D), _f32)] * 4,
    mesh=_mesh,
    scratch_types=_SC_SCRATCH,
)


def _tc1_body(h1, a101, a021, gw1, gb1, gw2, gb2, w021, vb021, gam, bet, out,
              *, relu):
  x = a101[...] + 1.1 * h1[...]
  hh = jnp.maximum(
      jnp.dot(x, gw1[...], preferred_element_type=_f32) + gb1[...], 0.0)
  hgin = jnp.dot(hh, gw2[...], preferred_element_type=_f32) + gb2[...]
  o021 = (jnp.dot(a021[...], w021[...], preferred_element_type=_f32)
          + vb021[...]) * 0.1
  y = (hgin + o021) * 0.5
  m = jnp.mean(y, axis=0, keepdims=True)
  v = jnp.mean((y - m) ** 2, axis=0, keepdims=True)
  y = gam[...] * (y - m) * lax.rsqrt(v + 1e-5) + bet[...]
  if relu:
    y = jnp.maximum(y, 0.0)
  out[...] = y


def _tc0_body(a110, a030, w110, vb110, w030, vb030, gam, bet, out, *, relu):
  o110 = (jnp.dot(a110[...], w110[...], preferred_element_type=_f32)
          + vb110[...]) * 0.1
  o030 = (jnp.dot(a030[...], w030[...], preferred_element_type=_f32)
          + vb030[...]) * 0.1
  y = (o110 + o030) * 0.5
  m = jnp.mean(y, axis=0, keepdims=True)
  v = jnp.mean((y - m) ** 2, axis=0, keepdims=True)
  y = gam[...] * (y - m) * lax.rsqrt(v + 1e-5) + bet[...]
  if relu:
    y = jnp.maximum(y, 0.0)
  out[...] = y


def _tc1(relu):
  return pl.pallas_call(
      functools.partial(_tc1_body, relu=relu),
      out_shape=jax.ShapeDtypeStruct((N, D), _f32))


def _tc0(relu):
  return pl.pallas_call(
      functools.partial(_tc0_body, relu=relu),
      out_shape=jax.ShapeDtypeStruct((N, D), _f32))


def _pad_src(v, total, mod):
  pad = total - v.shape[0]
  fill = jnp.arange(pad, dtype=_i32) % mod
  return jnp.concatenate([v.astype(_i32), fill]).reshape(NTILE, -1, 128)


def _pad_dst(v, total):
  pad = total - v.shape[0]
  fill = N + (jnp.arange(pad, dtype=_i32) % 16)
  return jnp.concatenate([v.astype(_i32), fill]).reshape(NTILE, -1, 128)


def kernel(x0, x1, edge_index_101, edge_attr_101, edge_index_110,
           edge_attr_110, edge_index_021, edge_attr_021, edge_index_030,
           edge_attr_030, x_emb1, x_emb2, edge_emb1, edge_emb2, gin_W1,
           gin_b1, gin_W2, gin_b2, W110, b110, W021, b021, W030, b030,
           bn_gamma, bn_beta):
  t360 = (x_emb1[:, None, :] + x_emb2[None, :, :]).reshape(360, D)
  t18 = (edge_emb1[:, None, :] + edge_emb2[None, :, :]).reshape(18, D)

  f0i = _pad_src(x0[:, 0] * 3 + x0[:, 1], NP, 360)
  f1i = _pad_src(x1[:, 0] * 3 + x1[:, 1], NP, 360)

  def eidx(ei, ea):
    fa = _pad_src(ea[:, 0] * 3 + ea[:, 1], EP, 18)
    s = _pad_src(ei[0], EP, N)
    dd = _pad_dst(ei[1], EP)
    return fa, s, dd

  fa101, s101, dd101 = eidx(edge_index_101, edge_attr_101)
  fa021, s021, dd021 = eidx(edge_index_021, edge_attr_021)
  fa110, s110, dd110 = eidx(edge_index_110, edge_attr_110)
  fa030, s030, dd030 = eidx(edge_index_030, edge_attr_030)

  h0p, h1p, p101, p021, p110, p030 = _sc_pre(
      t360, t18, f0i, f1i, fa101, dd101, fa021, dd021, fa110, dd110,
      fa030, dd030)
  h0 = h0p[:N]
  h1 = h1p[:N]

  gb1 = gin_b1.reshape(1, -1)
  gb2 = gin_b2.reshape(1, -1)
  vb110 = b110.reshape(1, -1)
  vb021 = b021.reshape(1, -1)
  vb030 = b030.reshape(1, -1)

  for layer in range(2):
    a101, a021, a110, a030 = _sc_layer(
        h0, h1, p101, p021, p110, p030, s101, dd101, s021, dd021,
        s110, dd110, s030, dd030)
    gam = bn_gamma[layer].reshape(1, D)
    bet = bn_beta[layer].reshape(1, D)
    relu = layer == 0
    h1 = _tc1(relu)(h1, a101, a021, gin_W1, gb1, gin_W2, gb2, W021, vb021,
                    gam, bet)
    h0 = _tc0(relu)(a110, a030, W110, vb110, W030, vb030, gam, bet)

  return jnp.concatenate([h0, h1], axis=0)


# speedup vs baseline: 6.5862x; 3.2599x over previous
import functools

import jax
import jax.numpy as jnp
from jax import lax
from jax.experimental import pallas as pl
from jax.experimental.pallas import tpu as pltpu
from jax.experimental.pallas import tpu_sc as plsc

N = 10000
D = 128
E = 160000
NTILE = 16
CH = 80
GCH = 16
EP = NTILE * CH * 128
CI = 5
NP = NTILE * CI * 128
ACC_ROWS = N + 16
RPT = 624
WB = 104
TAIL = N - NTILE * RPT

_mesh = plsc.VectorSubcoreMesh(core_axis_name="c", subcore_axis_name="s")
_f32 = jnp.float32
_i32 = jnp.int32


def _make_jobs(tile, src_v, dst_v, rows_a, rows_b, zbuf, acc, sem_a, sem_b):
  def init_job(table, fidx, outp):
    pltpu.sync_copy(fidx.at[tile], src_v.at[pl.ds(0, CI)])

    def body(j, carry):
      pltpu.async_copy(table.at[src_v.at[j]], rows_a, sem_a).wait()
      pltpu.sync_copy(rows_a, outp.at[pl.ds(tile * CI * 128 + j * 128, 128)])
      return carry

    lax.fori_loop(0, CI, body, 0)

  def seg_job(table, s2, d2, outp, base):
    if base is None:
      def zb(k, carry):
        pltpu.sync_copy(zbuf, acc.at[pl.ds(tile * RPT + k * 16, 16)])
        return carry
      lax.fori_loop(0, RPT // 16, zb, 0)

      @pl.when(tile == 0)
      def _():
        pltpu.sync_copy(zbuf, acc.at[pl.ds(NTILE * RPT, 16)])
    else:
      def ib(k, carry):
        off = tile * RPT + k * WB
        pltpu.sync_copy(base.at[pl.ds(off, WB)], rows_a.at[pl.ds(0, WB)])
        pltpu.sync_copy(rows_a.at[pl.ds(0, WB)], acc.at[pl.ds(off, WB)])
        return carry
      lax.fori_loop(0, RPT // WB, ib, 0)

      @pl.when(tile == 0)
      def _():
        pltpu.sync_copy(base.at[pl.ds(NTILE * RPT, TAIL)],
                        rows_a.at[pl.ds(0, TAIL)])
        pltpu.sync_copy(rows_a.at[pl.ds(0, TAIL)],
                        acc.at[pl.ds(NTILE * RPT, TAIL)])

    plsc.subcore_barrier()

    def grp(g, carry):
      pltpu.sync_copy(s2.at[tile, pl.ds(g * GCH, GCH)], src_v)
      pltpu.sync_copy(d2.at[tile, pl.ds(g * GCH, GCH)], dst_v)

      def eb(p, c2):
        j = 2 * p
        c0 = pltpu.async_copy(table.at[src_v.at[j]], rows_a, sem_a)
        c1 = pltpu.async_copy(table.at[src_v.at[j + 1]], rows_b, sem_b)
        c0.wait()
        pltpu.sync_copy(rows_a, acc.at[dst_v.at[j]], add=True)
        c1.wait()
        pltpu.sync_copy(rows_b, acc.at[dst_v.at[j + 1]], add=True)
        return c2

      lax.fori_loop(0, GCH // 2, eb, 0)
      return carry

    lax.fori_loop(0, CH // GCH, grp, 0)
    plsc.subcore_barrier()

    def ob(k, carry):
      off = tile * RPT + k * WB
      pltpu.sync_copy(acc.at[pl.ds(off, WB)], rows_a.at[pl.ds(0, WB)])
      pltpu.sync_copy(rows_a.at[pl.ds(0, WB)], outp.at[pl.ds(off, WB)])
      return carry

    lax.fori_loop(0, RPT // WB, ob, 0)

    @pl.when(tile == 0)
    def _():
      pltpu.sync_copy(acc.at[pl.ds(NTILE * RPT, TAIL)],
                      rows_b.at[pl.ds(0, TAIL)])
      pltpu.sync_copy(rows_b.at[pl.ds(0, TAIL)],
                      outp.at[pl.ds(NTILE * RPT, TAIL)])

  return init_job, seg_job


def _zero_zbuf(zbuf):
  z = jnp.zeros((16,), _f32)
  for r in range(16):
    for c in range(8):
      zbuf[r, pl.ds(16 * c, 16)] = z


def _pre_body(t360, t18, f0i, f1i, fa101, dd101, fa021, dd021, fa110, dd110,
              fa030, dd030, h0p, h1p, p101, p021, p110, p030,
              src_v, dst_v, rows_a, rows_b, zbuf, acc, sem_a, sem_b):
  tile = lax.axis_index("s")
  core = lax.axis_index("c")
  _zero_zbuf(zbuf)
  init_job, seg_job = _make_jobs(tile, src_v, dst_v, rows_a, rows_b, zbuf,
                                 acc, sem_a, sem_b)

  @pl.when(core == 0)
  def _():
    init_job(t360, f0i, h0p)
    seg_job(t18, fa101, dd101, p101, None)
    seg_job(t18, fa030, dd030, p030, None)

  @pl.when(core == 1)
  def _():
    init_job(t360, f1i, h1p)
    seg_job(t18, fa021, dd021, p021, None)
    seg_job(t18, fa110, dd110, p110, None)


def _layer_body(h0, h1, q101, q021, q110, q030, s101, dd101, s021, dd021,
                s110, dd110, s030, dd030, a101, a021, a110, a030,
                src_v, dst_v, rows_a, rows_b, zbuf, acc, sem_a, sem_b):
  tile = lax.axis_index("s")
  core = lax.axis_index("c")
  _, seg_job = _make_jobs(tile, src_v, dst_v, rows_a, rows_b, zbuf,
                          acc, sem_a, sem_b)

  @pl.when(core == 0)
  def _():
    seg_job(h1, s101, dd101, a101, q101)
    seg_job(h0, s030, dd030, a030, q030)

  @pl.when(core == 1)
  def _():
    seg_job(h0, s021, dd021, a021, q021)
    seg_job(h1, s110, dd110, a110, q110)


_SC_SCRATCH = [
    pltpu.VMEM((GCH, 128), _i32),
    pltpu.VMEM((GCH, 128), _i32),
    pltpu.VMEM((128, D), _f32),
    pltpu.VMEM((128, D), _f32),
    pltpu.VMEM((16, D), _f32),
    pltpu.VMEM_SHARED((ACC_ROWS, D), _f32),
    pltpu.SemaphoreType.DMA,
    pltpu.SemaphoreType.DMA,
]

_sc_pre = pl.kernel(
    _pre_body,
    out_type=[jax.ShapeDtypeStruct((NP, D), _f32),
              jax.ShapeDtypeStruct((NP, D), _f32)] +
             [jax.ShapeDtypeStruct((N, D), _f32)] * 4,
    mesh=_mesh,
    scratch_types=_SC_SCRATCH,
)

_sc_layer = pl.kernel(
    _layer_body,
    out_type=[jax.ShapeDtypeStruct((N, D), _f32)] * 4,
    mesh=_mesh,
    scratch_types=_SC_SCRATCH,
)


def _tc1_body(h1, a101, a021, gw1, gb1, gw2, gb2, w021, vb021, gam, bet, out,
              *, relu):
  x = a101[...] + 1.1 * h1[...]
  hh = jnp.maximum(
      jnp.dot(x, gw1[...], preferred_element_type=_f32) + gb1[...], 0.0)
  hgin = jnp.dot(hh, gw2[...], preferred_element_type=_f32) + gb2[...]
  o021 = (jnp.dot(a021[...], w021[...], preferred_element_type=_f32)
          + vb021[...]) * 0.1
  y = (hgin + o021) * 0.5
  m = jnp.mean(y, axis=0, keepdims=True)
  v = jnp.mean((y - m) ** 2, axis=0, keepdims=True)
  y = gam[...] * (y - m) * lax.rsqrt(v + 1e-5) + bet[...]
  if relu:
    y = jnp.maximum(y, 0.0)
  out[...] = y


def _tc0_body(a110, a030, w110, vb110, w030, vb030, gam, bet, out, *, relu):
  o110 = (jnp.dot(a110[...], w110[...], preferred_element_type=_f32)
          + vb110[...]) * 0.1
  o030 = (jnp.dot(a030[...], w030[...], preferred_element_type=_f32)
          + vb030[...]) * 0.1
  y = (o110 + o030) * 0.5
  m = jnp.mean(y, axis=0, keepdims=True)
  v = jnp.mean((y - m) ** 2, axis=0, keepdims=True)
  y = gam[...] * (y - m) * lax.rsqrt(v + 1e-5) + bet[...]
  if relu:
    y = jnp.maximum(y, 0.0)
  out[...] = y


def _tc1(relu):
  return pl.pallas_call(
      functools.partial(_tc1_body, relu=relu),
      out_shape=jax.ShapeDtypeStruct((N, D), _f32))


def _tc0(relu):
  return pl.pallas_call(
      functools.partial(_tc0_body, relu=relu),
      out_shape=jax.ShapeDtypeStruct((N, D), _f32))


def _pad_src(v, total, mod):
  pad = total - v.shape[0]
  fill = jnp.arange(pad, dtype=_i32) % mod
  return jnp.concatenate([v.astype(_i32), fill]).reshape(NTILE, -1, 128)


def _pad_dst(v, total):
  pad = total - v.shape[0]
  fill = N + (jnp.arange(pad, dtype=_i32) % 16)
  return jnp.concatenate([v.astype(_i32), fill]).reshape(NTILE, -1, 128)


def kernel(x0, x1, edge_index_101, edge_attr_101, edge_index_110,
           edge_attr_110, edge_index_021, edge_attr_021, edge_index_030,
           edge_attr_030, x_emb1, x_emb2, edge_emb1, edge_emb2, gin_W1,
           gin_b1, gin_W2, gin_b2, W110, b110, W021, b021, W030, b030,
           bn_gamma, bn_beta):
  R360, R18 = 28, 512
  t360 = jnp.tile((x_emb1[:, None, :] + x_emb2[None, :, :]).reshape(360, D),
                  (R360, 1))
  t18 = jnp.tile((edge_emb1[:, None, :] + edge_emb2[None, :, :]).reshape(18, D),
                 (R18, 1))

  def spread(f, nrows, nrep):
    return f + nrows * (jnp.arange(f.shape[0], dtype=_i32) % nrep)

  f0i = _pad_src(spread(x0[:, 0] * 3 + x0[:, 1], 360, R360), NP, 360 * R360)
  f1i = _pad_src(spread(x1[:, 0] * 3 + x1[:, 1], 360, R360), NP, 360 * R360)

  def eidx(ei, ea):
    fa = _pad_src(spread(ea[:, 0] * 3 + ea[:, 1], 18, R18), EP, 18 * R18)
    s = _pad_src(ei[0], EP, N)
    dd = _pad_dst(ei[1], EP)
    return fa, s, dd

  fa101, s101, dd101 = eidx(edge_index_101, edge_attr_101)
  fa021, s021, dd021 = eidx(edge_index_021, edge_attr_021)
  fa110, s110, dd110 = eidx(edge_index_110, edge_attr_110)
  fa030, s030, dd030 = eidx(edge_index_030, edge_attr_030)

  h0p, h1p, p101, p021, p110, p030 = _sc_pre(
      t360, t18, f0i, f1i, fa101, dd101, fa021, dd021, fa110, dd110,
      fa030, dd030)
  h0 = h0p[:N]
  h1 = h1p[:N]

  gb1 = gin_b1.reshape(1, -1)
  gb2 = gin_b2.reshape(1, -1)
  vb110 = b110.reshape(1, -1)
  vb021 = b021.reshape(1, -1)
  vb030 = b030.reshape(1, -1)

  for layer in range(2):
    a101, a021, a110, a030 = _sc_layer(
        h0, h1, p101, p021, p110, p030, s101, dd101, s021, dd021,
        s110, dd110, s030, dd030)
    gam = bn_gamma[layer].reshape(1, D)
    bet = bn_beta[layer].reshape(1, D)
    relu = layer == 0
    h1 = _tc1(relu)(h1, a101, a021, gin_W1, gb1, gin_W2, gb2, W021, vb021,
                    gam, bet)
    h0 = _tc0(relu)(a110, a030, W110, vb110, W030, vb030, gam, bet)

  return jnp.concatenate([h0, h1], axis=0)
